# baseline (speedup 1.0000x reference)
import jax
import jax.numpy as jnp
from jax.experimental import pallas as pl
from jax.experimental.pallas import tpu as pltpu

N = 10000
H = 128
NCLASSES = 40
BM = 200
NI = N // BM

_HI = jax.lax.Precision.HIGHEST


def _in_mlp_kernel(f_ref, w1_ref, b1_ref, of_ref, ob_ref):
    x = jax.nn.relu(
        jnp.dot(f_ref[...], w1_ref[...], precision=_HI) + b1_ref[...]
    )
    of_ref[...] = x
    ob_ref[...] = x.astype(jnp.bfloat16)


def _gru_ginmlp(y, h, wih, whh, bih, bhh, wg1, bg1, wg2, bg2):
    gi = jnp.dot(y, wih, precision=_HI) + bih
    gh = jnp.dot(h, whh, precision=_HI) + bhh
    r = jax.nn.sigmoid(gi[:, :H] + gh[:, :H])
    z = jax.nn.sigmoid(gi[:, H:2 * H] + gh[:, H:2 * H])
    n = jnp.tanh(gi[:, 2 * H:] + r * gh[:, 2 * H:])
    hn = (1.0 - z) * n + z * h
    g = jnp.dot(jax.nn.relu(jnp.dot(hn, wg1, precision=_HI) + bg1),
                wg2, precision=_HI) + bg2
    return jax.nn.relu(g)


def _layer_kernel(adj_ref, xb_ref, h_ref, wih_ref, whh_ref, bih_ref,
                  bhh_ref, wg1_ref, bg1_ref, wg2_ref, bg2_ref,
                  of_ref, ob_ref):
    a = adj_ref[...].astype(jnp.bfloat16)
    y = jnp.dot(a, xb_ref[...], preferred_element_type=jnp.float32)
    out = _gru_ginmlp(y, h_ref[...], wih_ref[...], whh_ref[...],
                      bih_ref[...], bhh_ref[...], wg1_ref[...],
                      bg1_ref[...], wg2_ref[...], bg2_ref[...])
    of_ref[...] = out
    ob_ref[...] = out.astype(jnp.bfloat16)


def _final_kernel(adj_ref, xb_ref, h_ref, wih_ref, whh_ref, bih_ref,
                  bhh_ref, wg1_ref, bg1_ref, wg2_ref, bg2_ref,
                  wc_ref, bc_ref, wd_ref, bd_ref, pred_ref):
    a = adj_ref[...].astype(jnp.bfloat16)
    y = jnp.dot(a, xb_ref[...], preferred_element_type=jnp.float32)
    out = _gru_ginmlp(y, h_ref[...], wih_ref[...], whh_ref[...],
                      bih_ref[...], bhh_ref[...], wg1_ref[...],
                      bg1_ref[...], wg2_ref[...], bg2_ref[...])
    t = jnp.dot(jax.nn.relu(jnp.dot(out, wc_ref[...], precision=_HI)
                            + bc_ref[...]),
                wd_ref[...], precision=_HI) + bd_ref[...]
    m = jnp.max(t, axis=1, keepdims=True)
    e = jnp.exp(t - m)
    pred_ref[...] = e / jnp.sum(e, axis=1, keepdims=True)


def _full(shape):
    return pl.BlockSpec(shape, lambda i: (0,) * len(shape))


def _layer_call(adj, x_f32, x_bf16, wih_t, whh_t, bih, bhh, wg1, bg1,
                wg2, bg2, head=None):
    weight_specs = [
        _full((H, 3 * H)), _full((H, 3 * H)), _full((1, 3 * H)),
        _full((1, 3 * H)), _full((H, H)), _full((1, H)),
        _full((H, H)), _full((1, H)),
    ]
    in_specs = [
        pl.BlockSpec((BM, N), lambda i: (i, 0)),
        _full((N, H)),
        pl.BlockSpec((BM, H), lambda i: (i, 0)),
    ] + weight_specs
    args = [adj, x_bf16, x_f32, wih_t, whh_t, bih, bhh, wg1, bg1, wg2, bg2]
    if head is None:
        body = _layer_kernel
        out_shape = (jax.ShapeDtypeStruct((N, H), jnp.float32),
                     jax.ShapeDtypeStruct((N, H), jnp.bfloat16))
        out_specs = (pl.BlockSpec((BM, H), lambda i: (i, 0)),
                     pl.BlockSpec((BM, H), lambda i: (i, 0)))
    else:
        body = _final_kernel
        wc, bc, wd, bd = head
        in_specs += [_full((H, H)), _full((1, H)),
                     _full((H, NCLASSES)), _full((1, NCLASSES))]
        args += [wc, bc, wd, bd]
        out_shape = jax.ShapeDtypeStruct((N, NCLASSES), jnp.float32)
        out_specs = pl.BlockSpec((BM, NCLASSES), lambda i: (i, 0))
    return pl.pallas_call(
        body,
        grid=(NI,),
        in_specs=in_specs,
        out_specs=out_specs,
        out_shape=out_shape,
        compiler_params=pltpu.CompilerParams(
            dimension_semantics=("arbitrary",)),
    )(*args)


def kernel(features, adj, W1, b1, Wih, Whh, bih, bhh, Wg1, bg1, Wg2, bg2,
           Wc, bc, Wd, bd):
    x_f32, x_bf16 = pl.pallas_call(
        _in_mlp_kernel,
        grid=(NI,),
        in_specs=[pl.BlockSpec((BM, H), lambda i: (i, 0)),
                  pl.BlockSpec((H, H), lambda i: (0, 0)),
                  pl.BlockSpec((1, H), lambda i: (0, 0))],
        out_specs=(pl.BlockSpec((BM, H), lambda i: (i, 0)),
                   pl.BlockSpec((BM, H), lambda i: (i, 0))),
        out_shape=(jax.ShapeDtypeStruct((N, H), jnp.float32),
                   jax.ShapeDtypeStruct((N, H), jnp.bfloat16)),
        compiler_params=pltpu.CompilerParams(
            dimension_semantics=("arbitrary",)),
    )(features, W1, b1.reshape(1, H))

    for i in range(2):
        layer = dict(
            wih_t=Wih[i].T, whh_t=Whh[i].T,
            bih=bih[i].reshape(1, 3 * H), bhh=bhh[i].reshape(1, 3 * H),
            wg1=Wg1[i], bg1=bg1[i].reshape(1, H),
            wg2=Wg2[i], bg2=bg2[i].reshape(1, H),
        )
        if i == 0:
            x_f32, x_bf16 = _layer_call(adj, x_f32, x_bf16, **layer)
        else:
            preds = _layer_call(
                adj, x_f32, x_bf16, **layer,
                head=(Wc, bc.reshape(1, H), Wd, bd.reshape(1, NCLASSES)))
    return preds


# speedup vs baseline: 1.4271x; 1.4271x over previous
import jax
import jax.numpy as jnp
from jax.experimental import pallas as pl
from jax.experimental.pallas import tpu as pltpu

N = 10000
H = 128
NCLASSES = 40
BM = 400
NI = N // BM

_HI = jax.lax.Precision.DEFAULT


def _in_mlp_kernel(f_ref, w1_ref, b1_ref, o_ref):
    o_ref[...] = jax.nn.relu(
        jnp.dot(f_ref[...], w1_ref[...], precision=_HI) + b1_ref[...]
    )


def _gru_ginmlp(y, h, wih, whh, bih, bhh, wg1, bg1, wg2, bg2):
    gi = jnp.dot(y, wih, precision=_HI) + bih
    gh = jnp.dot(h, whh, precision=_HI) + bhh
    r = jax.nn.sigmoid(gi[:, :H] + gh[:, :H])
    z = jax.nn.sigmoid(gi[:, H:2 * H] + gh[:, H:2 * H])
    n = jnp.tanh(gi[:, 2 * H:] + r * gh[:, 2 * H:])
    hn = (1.0 - z) * n + z * h
    g = jnp.dot(jax.nn.relu(jnp.dot(hn, wg1, precision=_HI) + bg1),
                wg2, precision=_HI) + bg2
    return jax.nn.relu(g)


def _layer_kernel(adj_ref, x_ref, wih_ref, whh_ref, bih_ref,
                  bhh_ref, wg1_ref, bg1_ref, wg2_ref, bg2_ref,
                  o_ref):
    i = pl.program_id(0)
    y = jnp.dot(adj_ref[...], x_ref[...], precision=_HI,
                preferred_element_type=jnp.float32)
    h = x_ref[pl.ds(i * BM, BM), :]
    o_ref[...] = _gru_ginmlp(y, h, wih_ref[...], whh_ref[...],
                             bih_ref[...], bhh_ref[...], wg1_ref[...],
                             bg1_ref[...], wg2_ref[...], bg2_ref[...])


def _final_kernel(adj_ref, x_ref, wih_ref, whh_ref, bih_ref,
                  bhh_ref, wg1_ref, bg1_ref, wg2_ref, bg2_ref,
                  wc_ref, bc_ref, wd_ref, bd_ref, pred_ref):
    i = pl.program_id(0)
    y = jnp.dot(adj_ref[...], x_ref[...], precision=_HI,
                preferred_element_type=jnp.float32)
    h = x_ref[pl.ds(i * BM, BM), :]
    out = _gru_ginmlp(y, h, wih_ref[...], whh_ref[...],
                      bih_ref[...], bhh_ref[...], wg1_ref[...],
                      bg1_ref[...], wg2_ref[...], bg2_ref[...])
    t = jnp.dot(jax.nn.relu(jnp.dot(out, wc_ref[...], precision=_HI)
                            + bc_ref[...]),
                wd_ref[...], precision=_HI) + bd_ref[...]
    m = jnp.max(t, axis=1, keepdims=True)
    e = jnp.exp(t - m)
    pred_ref[...] = e / jnp.sum(e, axis=1, keepdims=True)


def _full(shape):
    return pl.BlockSpec(shape, lambda i: (0,) * len(shape))


def _layer_call(adj, x, wih_t, whh_t, bih, bhh, wg1, bg1,
                wg2, bg2, head=None):
    weight_specs = [
        _full((H, 3 * H)), _full((H, 3 * H)), _full((1, 3 * H)),
        _full((1, 3 * H)), _full((H, H)), _full((1, H)),
        _full((H, H)), _full((1, H)),
    ]
    in_specs = [
        pl.BlockSpec((BM, N), lambda i: (i, 0)),
        _full((N, H)),
    ] + weight_specs
    args = [adj, x, wih_t, whh_t, bih, bhh, wg1, bg1, wg2, bg2]
    if head is None:
        body = _layer_kernel
        out_shape = jax.ShapeDtypeStruct((N, H), jnp.float32)
        out_specs = pl.BlockSpec((BM, H), lambda i: (i, 0))
    else:
        body = _final_kernel
        wc, bc, wd, bd = head
        in_specs += [_full((H, H)), _full((1, H)),
                     _full((H, NCLASSES)), _full((1, NCLASSES))]
        args += [wc, bc, wd, bd]
        out_shape = jax.ShapeDtypeStruct((N, NCLASSES), jnp.float32)
        out_specs = pl.BlockSpec((BM, NCLASSES), lambda i: (i, 0))
    return pl.pallas_call(
        body,
        grid=(NI,),
        in_specs=in_specs,
        out_specs=out_specs,
        out_shape=out_shape,
        compiler_params=pltpu.CompilerParams(
            dimension_semantics=("arbitrary",)),
    )(*args)


def kernel(features, adj, W1, b1, Wih, Whh, bih, bhh, Wg1, bg1, Wg2, bg2,
           Wc, bc, Wd, bd):
    x = pl.pallas_call(
        _in_mlp_kernel,
        grid=(NI,),
        in_specs=[pl.BlockSpec((BM, H), lambda i: (i, 0)),
                  pl.BlockSpec((H, H), lambda i: (0, 0)),
                  pl.BlockSpec((1, H), lambda i: (0, 0))],
        out_specs=pl.BlockSpec((BM, H), lambda i: (i, 0)),
        out_shape=jax.ShapeDtypeStruct((N, H), jnp.float32),
        compiler_params=pltpu.CompilerParams(
            dimension_semantics=("arbitrary",)),
    )(features, W1, b1.reshape(1, H))

    for i in range(2):
        layer = dict(
            wih_t=Wih[i].T, whh_t=Whh[i].T,
            bih=bih[i].reshape(1, 3 * H), bhh=bhh[i].reshape(1, 3 * H),
            wg1=Wg1[i], bg1=bg1[i].reshape(1, H),
            wg2=Wg2[i], bg2=bg2[i].reshape(1, H),
        )
        if i == 0:
            x = _layer_call(adj, x, **layer)
        else:
            preds = _layer_call(
                adj, x, **layer,
                head=(Wc, bc.reshape(1, H), Wd, bd.reshape(1, NCLASSES)))
    return preds
